# baseline (speedup 1.0000x reference)
import functools

import jax
import jax.numpy as jnp
from jax import lax
from jax.experimental import pallas as pl
from jax.experimental.pallas import tpu as pltpu
from jax.experimental.pallas import tpu_sc as plsc

F = 26
V = 100000
L = 16
B = 16384
H = 64

NC = 2
NS = 16
NW = NC * NS
BPW = B // NW
CH = 64
IDX_PER_CH = CH * F
IDX_ROWS = IDX_PER_CH // 128
CHUNKS = BPW // CH


def _gather_sum_body(emb_hbm, idx_hbm, out_hbm, idx_v, rows_v, h_v, sem):
    wid = lax.axis_index("s") * NC + lax.axis_index("c")

    pltpu.sync_copy(idx_hbm.at[pl.ds(wid * (IDX_ROWS * CHUNKS), IDX_ROWS * CHUNKS)],
                    idx_v)

    def chunk_body(g, carry):
        copies = [
            pltpu.async_copy(
                emb_hbm.at[idx_v.at[g * IDX_ROWS + j]],
                rows_v.at[pl.ds(j * 128, 128)],
                sem,
            )
            for j in range(IDX_ROWS)
        ]
        for c in copies:
            c.wait()

        def sum_body(b, carry2):
            base = b * F
            acc = rows_v[base]
            for f in range(1, F):
                acc = acc + rows_v[base + f]
            h_v[b] = acc
            return carry2

        lax.fori_loop(0, CH, sum_body, 0, unroll=2)

        pltpu.sync_copy(h_v, out_hbm.at[pl.ds(wid * BPW + g * CH, CH)])
        return carry

    lax.fori_loop(0, CHUNKS, chunk_body, 0)


def _gather_sum(emb_flat, idx2d):
    mesh = plsc.VectorSubcoreMesh(
        core_axis_name="c", subcore_axis_name="s", num_cores=NC, num_subcores=NS)
    return pl.kernel(
        _gather_sum_body,
        out_type=jax.ShapeDtypeStruct((B, L), jnp.float32),
        mesh=mesh,
        scratch_types=[
            pltpu.VMEM((IDX_ROWS * CHUNKS, 128), jnp.int32),
            pltpu.VMEM((IDX_PER_CH, L), jnp.float32),
            pltpu.VMEM((CH, L), jnp.float32),
            pltpu.SemaphoreType.DMA,
        ],
        compiler_params=pltpu.CompilerParams(use_tc_tiling_on_sc=False),
    )(emb_flat, idx2d)


TR_A, TR_CD = 40, 2500
TR_NJ = TR_A // 8


def _tr_body(in_ref, out_ref):
    for r in range(8):
        out_ref[pl.ds(r * TR_CD, TR_CD), :] = in_ref[0, :, r, :].T


def _transpose_tables(emb_t):
    emb4 = emb_t.reshape(F, L, TR_A, TR_CD)
    return pl.pallas_call(
        _tr_body,
        grid=(F, TR_NJ),
        in_specs=[pl.BlockSpec((1, L, 8, TR_CD), lambda f, j: (f, 0, j, 0))],
        out_specs=pl.BlockSpec((8 * TR_CD, L), lambda f, j: (f * TR_NJ + j, 0)),
        out_shape=jax.ShapeDtypeStruct((F * V, L), jnp.float32),
    )(emb4)


MLP_BLK = 2048


def _mlp_body(h_ref, w1_ref, b1_ref, w2_ref, b2_ref, w3_ref, b3_ref, out_ref):
    x = h_ref[...]
    x = jnp.maximum(
        jnp.dot(x, w1_ref[...], preferred_element_type=jnp.float32) + b1_ref[...], 0.0)
    x = jnp.maximum(
        jnp.dot(x, w2_ref[...], preferred_element_type=jnp.float32) + b2_ref[...], 0.0)
    out_ref[...] = jnp.maximum(
        jnp.dot(x, w3_ref[...], preferred_element_type=jnp.float32) + b3_ref[...], 0.0)


def _mlp(h, W1, b1, W2, b2, W3, b3):
    full = lambda s: pl.BlockSpec(s, lambda i: (0, 0))
    return pl.pallas_call(
        _mlp_body,
        grid=(B // MLP_BLK,),
        in_specs=[
            pl.BlockSpec((MLP_BLK, L), lambda i: (i, 0)),
            full(W1.shape), full((1, L)),
            full(W2.shape), full((1, 2 * L)),
            full(W3.shape), full((1, H)),
        ],
        out_specs=pl.BlockSpec((MLP_BLK, H), lambda i: (i, 0)),
        out_shape=jax.ShapeDtypeStruct((B, H), jnp.float32),
    )(h, W1, b1.reshape(1, L), W2, b2.reshape(1, 2 * L), W3, b3.reshape(1, H))


def kernel(mof, emb, W1, b1, W2, b2, W3, b3):
    offs = (jnp.arange(F, dtype=jnp.int32) * V)[None, :]
    flat_idx = (mof.astype(jnp.int32) + offs).reshape(-1, 128)
    emb_flat = _transpose_tables(emb.transpose(0, 2, 1))
    h = _gather_sum(emb_flat, flat_idx)
    return _mlp(h, W1, b1, W2, b2, W3, b3)


# speedup vs baseline: 3.5302x; 3.5302x over previous
import functools

import jax
import jax.numpy as jnp
from jax import lax
from jax.experimental import pallas as pl
from jax.experimental.pallas import tpu as pltpu
from jax.experimental.pallas import tpu_sc as plsc

F = 26
V = 100000
L = 16
B = 16384
H = 64

NC = 2
NS = 16
NW = NC * NS
BPW = B // NW
CH = 64
IDX_PER_CH = CH * F
IDX_ROWS = IDX_PER_CH // 128
CHUNKS = BPW // CH


def _gather_sum_body(emb_hbm, idx_hbm, out_hbm, idx_v, rows_v, h_v, sem):
    wid = lax.axis_index("s") * NC + lax.axis_index("c")

    pltpu.sync_copy(idx_hbm.at[pl.ds(wid * (IDX_ROWS * CHUNKS), IDX_ROWS * CHUNKS)],
                    idx_v)

    def chunk_body(g, carry):
        copies = [
            pltpu.async_copy(
                emb_hbm.at[idx_v.at[g * IDX_ROWS + j]],
                rows_v.at[pl.ds(j * 128, 128)],
                sem,
            )
            for j in range(IDX_ROWS)
        ]
        for c in copies:
            c.wait()

        def sum_body(b, carry2):
            base = b * F
            acc = rows_v[base]
            for f in range(1, F):
                acc = acc + rows_v[base + f]
            h_v[b] = acc
            return carry2

        lax.fori_loop(0, CH, sum_body, 0, unroll=2)

        pltpu.sync_copy(h_v, out_hbm.at[pl.ds(wid * BPW + g * CH, CH)])
        return carry

    lax.fori_loop(0, CHUNKS, chunk_body, 0)


def _gather_sum(emb_flat, idx2d):
    mesh = plsc.VectorSubcoreMesh(
        core_axis_name="c", subcore_axis_name="s", num_cores=NC, num_subcores=NS)
    return pl.kernel(
        _gather_sum_body,
        out_type=jax.ShapeDtypeStruct((B, L), jnp.float32),
        mesh=mesh,
        scratch_types=[
            pltpu.VMEM((IDX_ROWS * CHUNKS, 128), jnp.int32),
            pltpu.VMEM((IDX_PER_CH, L), jnp.float32),
            pltpu.VMEM((CH, L), jnp.float32),
            pltpu.SemaphoreType.DMA,
        ],
        compiler_params=pltpu.CompilerParams(use_tc_tiling_on_sc=False),
    )(emb_flat, idx2d)


VQ = V // 8


_TR_CHUNKS = [(0, 2560), (2560, 2560), (5120, 2560), (7680, 2560), (10240, 2260)]


def _tr_body(x_ref, p_ref, out_ref):
    p = p_ref[...]
    for t in range(2):
        for off, w in _TR_CHUNKS:
            xc = x_ref[0, pl.ds(t * L, L), :, pl.ds(off, w)]
            xr = xc.reshape(8 * L, w)
            z = jax.lax.dot_general(
                xr, p, (((0,), (0,)), ((), ())),
                preferred_element_type=jnp.float32,
                precision=jax.lax.Precision.HIGHEST)
            out_ref[0, pl.ds(t * VQ + off, w), :] = z


def _transpose_tables(emb_t):
    x = emb_t.reshape(F // 2, 2 * L, 8, VQ)
    perm = jnp.zeros((128, 128), jnp.float32).at[
        jnp.arange(128), (jnp.arange(128) % 8) * 16 + jnp.arange(128) // 8
    ].set(1.0)
    out = pl.pallas_call(
        _tr_body,
        grid=(F // 2,),
        in_specs=[
            pl.BlockSpec((1, 2 * L, 8, VQ), lambda f: (f, 0, 0, 0)),
            pl.BlockSpec((128, 128), lambda f: (0, 0)),
        ],
        out_specs=pl.BlockSpec((1, 2 * VQ, 128), lambda f: (f, 0, 0)),
        out_shape=jax.ShapeDtypeStruct((F // 2, 2 * VQ, 128), jnp.float32),
        compiler_params=pltpu.CompilerParams(vmem_limit_bytes=110 * 2**20),
    )(x, perm)
    return out.reshape(F * V, L)


MLP_BLK = 2048


def _mlp_body(h_ref, w1_ref, b1_ref, w2_ref, b2_ref, w3_ref, b3_ref, out_ref):
    x = h_ref[...]
    x = jnp.maximum(
        jnp.dot(x, w1_ref[...], preferred_element_type=jnp.float32) + b1_ref[...], 0.0)
    x = jnp.maximum(
        jnp.dot(x, w2_ref[...], preferred_element_type=jnp.float32) + b2_ref[...], 0.0)
    out_ref[...] = jnp.maximum(
        jnp.dot(x, w3_ref[...], preferred_element_type=jnp.float32) + b3_ref[...], 0.0)


def _mlp(h, W1, b1, W2, b2, W3, b3):
    full = lambda s: pl.BlockSpec(s, lambda i: (0, 0))
    return pl.pallas_call(
        _mlp_body,
        grid=(B // MLP_BLK,),
        in_specs=[
            pl.BlockSpec((MLP_BLK, L), lambda i: (i, 0)),
            full(W1.shape), full((1, L)),
            full(W2.shape), full((1, 2 * L)),
            full(W3.shape), full((1, H)),
        ],
        out_specs=pl.BlockSpec((MLP_BLK, H), lambda i: (i, 0)),
        out_shape=jax.ShapeDtypeStruct((B, H), jnp.float32),
    )(h, W1, b1.reshape(1, L), W2, b2.reshape(1, 2 * L), W3, b3.reshape(1, H))


def kernel(mof, emb, W1, b1, W2, b2, W3, b3):
    offs = (jnp.arange(F, dtype=jnp.int32) * V)[None, :]
    v = mof.astype(jnp.int32)
    flat_idx = ((v % VQ) * 8 + v // VQ + offs).reshape(-1, 128)
    emb_flat = _transpose_tables(emb.transpose(0, 2, 1))
    h = _gather_sum(emb_flat, flat_idx)
    return _mlp(h, W1, b1, W2, b2, W3, b3)


# speedup vs baseline: 3.6163x; 1.0244x over previous
import functools

import jax
import jax.numpy as jnp
from jax import lax
from jax.experimental import pallas as pl
from jax.experimental.pallas import tpu as pltpu
from jax.experimental.pallas import tpu_sc as plsc

F = 26
V = 100000
L = 16
B = 16384
H = 64

NC = 2
NS = 16
NW = NC * NS
BPW = B // NW
CH = 64
IDX_PER_CH = CH * F
IDX_ROWS = IDX_PER_CH // 128
CHUNKS = BPW // CH


def _gather_sum_body(emb_hbm, idx_hbm, out_hbm, idx_v, rows_v, h_v, sem):
    wid = lax.axis_index("s") * NC + lax.axis_index("c")

    pltpu.sync_copy(idx_hbm.at[pl.ds(wid * (IDX_ROWS * CHUNKS), IDX_ROWS * CHUNKS)],
                    idx_v)

    def chunk_body(g, carry):
        copies = [
            pltpu.async_copy(
                emb_hbm.at[idx_v.at[g * IDX_ROWS + j]],
                rows_v.at[pl.ds(j * 128, 128)],
                sem,
            )
            for j in range(IDX_ROWS)
        ]
        for c in copies:
            c.wait()

        def sum_body(b, carry2):
            base = b * F
            acc = rows_v[base]
            for f in range(1, F):
                acc = acc + rows_v[base + f]
            h_v[b] = acc
            return carry2

        lax.fori_loop(0, CH, sum_body, 0, unroll=2)

        pltpu.sync_copy(h_v, out_hbm.at[pl.ds(wid * BPW + g * CH, CH)])
        return carry

    lax.fori_loop(0, CHUNKS, chunk_body, 0)


def _gather_sum(emb_flat, idx2d):
    mesh = plsc.VectorSubcoreMesh(
        core_axis_name="c", subcore_axis_name="s", num_cores=NC, num_subcores=NS)
    return pl.kernel(
        _gather_sum_body,
        out_type=jax.ShapeDtypeStruct((B, L), jnp.float32),
        mesh=mesh,
        scratch_types=[
            pltpu.VMEM((IDX_ROWS * CHUNKS, 128), jnp.int32),
            pltpu.VMEM((IDX_PER_CH, L), jnp.float32),
            pltpu.VMEM((CH, L), jnp.float32),
            pltpu.SemaphoreType.DMA,
        ],
        compiler_params=pltpu.CompilerParams(use_tc_tiling_on_sc=False),
    )(emb_flat, idx2d)


VQ = V // 8


_TR_CHUNKS = [(i * 1280, 1280) for i in range(9)] + [(11520, 980)]


def _tr_body(x_ref, p_ref, out_ref):
    p = p_ref[...]
    for off, w in _TR_CHUNKS:
        xc = x_ref[0, :, :, pl.ds(off, w)]
        xr = xc.reshape(16 * L, w)
        z = jax.lax.dot_general(
            xr, p, (((0,), (0,)), ((), ())),
            preferred_element_type=jnp.float32,
            precision=jax.lax.Precision.HIGHEST)
        for t in range(2):
            out_ref[0, pl.ds(t * VQ + off, w), :] = z[:, t * 128:(t + 1) * 128]


def _transpose_tables(emb_t):
    x = emb_t.reshape(F // 2, 2 * L, 8, VQ)
    r = jnp.arange(256)
    rr = r % 128
    perm = jnp.zeros((256, 256), jnp.float32).at[
        r, (r // 128) * 128 + (rr % 8) * 16 + rr // 8
    ].set(1.0)
    out = pl.pallas_call(
        _tr_body,
        grid=(F // 2,),
        in_specs=[
            pl.BlockSpec((1, 2 * L, 8, VQ), lambda f: (f, 0, 0, 0)),
            pl.BlockSpec((256, 256), lambda f: (0, 0)),
        ],
        out_specs=pl.BlockSpec((1, 2 * VQ, 128), lambda f: (f, 0, 0)),
        out_shape=jax.ShapeDtypeStruct((F // 2, 2 * VQ, 128), jnp.float32),
        compiler_params=pltpu.CompilerParams(vmem_limit_bytes=110 * 2**20),
    )(x, perm)
    return out.reshape(F * V, L)


MLP_BLK = 2048


def _mlp_body(h_ref, w1_ref, b1_ref, w2_ref, b2_ref, w3_ref, b3_ref, out_ref):
    x = h_ref[...]
    x = jnp.maximum(
        jnp.dot(x, w1_ref[...], preferred_element_type=jnp.float32) + b1_ref[...], 0.0)
    x = jnp.maximum(
        jnp.dot(x, w2_ref[...], preferred_element_type=jnp.float32) + b2_ref[...], 0.0)
    out_ref[...] = jnp.maximum(
        jnp.dot(x, w3_ref[...], preferred_element_type=jnp.float32) + b3_ref[...], 0.0)


def _mlp(h, W1, b1, W2, b2, W3, b3):
    full = lambda s: pl.BlockSpec(s, lambda i: (0, 0))
    return pl.pallas_call(
        _mlp_body,
        grid=(B // MLP_BLK,),
        in_specs=[
            pl.BlockSpec((MLP_BLK, L), lambda i: (i, 0)),
            full(W1.shape), full((1, L)),
            full(W2.shape), full((1, 2 * L)),
            full(W3.shape), full((1, H)),
        ],
        out_specs=pl.BlockSpec((MLP_BLK, H), lambda i: (i, 0)),
        out_shape=jax.ShapeDtypeStruct((B, H), jnp.float32),
    )(h, W1, b1.reshape(1, L), W2, b2.reshape(1, 2 * L), W3, b3.reshape(1, H))


def kernel(mof, emb, W1, b1, W2, b2, W3, b3):
    offs = (jnp.arange(F, dtype=jnp.int32) * V)[None, :]
    v = mof.astype(jnp.int32)
    flat_idx = ((v % VQ) * 8 + v // VQ + offs).reshape(-1, 128)
    emb_flat = _transpose_tables(emb.transpose(0, 2, 1))
    h = _gather_sum(emb_flat, flat_idx)
    return _mlp(h, W1, b1, W2, b2, W3, b3)


# speedup vs baseline: 5.9194x; 1.6368x over previous
import functools

import jax
import jax.numpy as jnp
import numpy as np
from jax import lax
from jax.experimental import pallas as pl
from jax.experimental.pallas import tpu as pltpu
from jax.experimental.pallas import tpu_sc as plsc

F = 26
V = 100000
L = 16
B = 16384
H = 64

NC = 2
NS = 16
NW = NC * NS
BPW = B // NW
CH = 64
IDX_PER_CH = CH * F
IDX_ROWS = IDX_PER_CH // 128
CHUNKS = BPW // CH


def _gather_sum_body(emb_hbm, idx_hbm, out_hbm, idx_v, rows_v, h_v, sem):
    wid = lax.axis_index("s") * NC + lax.axis_index("c")

    pltpu.sync_copy(idx_hbm.at[pl.ds(wid * (IDX_ROWS * CHUNKS), IDX_ROWS * CHUNKS)],
                    idx_v)

    def chunk_body(g, carry):
        copies = [
            pltpu.async_copy(
                emb_hbm.at[idx_v.at[g * IDX_ROWS + j]],
                rows_v.at[pl.ds(j * 128, 128)],
                sem,
            )
            for j in range(IDX_ROWS)
        ]
        for c in copies:
            c.wait()

        def sum_body(b, carry2):
            base = b * F
            acc = rows_v[base]
            for f in range(1, F):
                acc = acc + rows_v[base + f]
            h_v[b] = acc
            return carry2

        lax.fori_loop(0, CH, sum_body, 0, unroll=2)

        pltpu.sync_copy(h_v, out_hbm.at[pl.ds(wid * BPW + g * CH, CH)])
        return carry

    lax.fori_loop(0, CHUNKS, chunk_body, 0)


def _gather_sum(emb_flat, idx2d):
    mesh = plsc.VectorSubcoreMesh(
        core_axis_name="c", subcore_axis_name="s", num_cores=NC, num_subcores=NS)
    return pl.kernel(
        _gather_sum_body,
        out_type=jax.ShapeDtypeStruct((B, L), jnp.float32),
        mesh=mesh,
        scratch_types=[
            pltpu.VMEM((IDX_ROWS * CHUNKS, 128), jnp.int32),
            pltpu.VMEM((IDX_PER_CH, L), jnp.float32),
            pltpu.VMEM((CH, L), jnp.float32),
            pltpu.SemaphoreType.DMA,
        ],
        compiler_params=pltpu.CompilerParams(use_tc_tiling_on_sc=False),
    )(emb_flat, idx2d)


VQ = V // 8


_TR_CHUNKS = [(i * 1280, 1280) for i in range(9)] + [(11520, 980)]

_PR = np.arange(256)
_PERM = np.zeros((256, 256), np.float32)
_PERM[_PR, (_PR % 32 // 16) * 128 + (_PR // 32) * 16 + _PR % 16] = 1.0


def _tr_body(x_ref, p_ref, out_ref):
    p = p_ref[...]
    for off, w in _TR_CHUNKS:
        xq = jnp.concatenate(
            [x_ref[:, :, pl.ds(q * VQ + off, w)] for q in range(8)], axis=0)
        xr = xq.reshape(256, w)
        z = jax.lax.dot_general(
            xr, p, (((0,), (0,)), ((), ())),
            preferred_element_type=jnp.float32,
            precision=jax.lax.Precision.HIGHEST)
        for t in range(2):
            out_ref[0, pl.ds(t * VQ + off, w), :] = z[:, t * 128:(t + 1) * 128]


def _transpose_tables(emb_t):
    out = pl.pallas_call(
        _tr_body,
        grid=(F // 2,),
        in_specs=[
            pl.BlockSpec((2, L, V), lambda f: (f, 0, 0)),
            pl.BlockSpec((256, 256), lambda f: (0, 0)),
        ],
        out_specs=pl.BlockSpec((1, 2 * VQ, 128), lambda f: (f, 0, 0)),
        out_shape=jax.ShapeDtypeStruct((F // 2, 2 * VQ, 128), jnp.float32),
        compiler_params=pltpu.CompilerParams(vmem_limit_bytes=110 * 2**20),
    )(emb_t, jnp.asarray(_PERM))
    return out.reshape(F * V, L)


MLP_BLK = 2048


def _mlp_body(h_ref, w1_ref, b1_ref, w2_ref, b2_ref, w3_ref, b3_ref, out_ref):
    x = h_ref[...]
    x = jnp.maximum(
        jnp.dot(x, w1_ref[...], preferred_element_type=jnp.float32) + b1_ref[...], 0.0)
    x = jnp.maximum(
        jnp.dot(x, w2_ref[...], preferred_element_type=jnp.float32) + b2_ref[...], 0.0)
    out_ref[...] = jnp.maximum(
        jnp.dot(x, w3_ref[...], preferred_element_type=jnp.float32) + b3_ref[...], 0.0)


def _mlp(h, W1, b1, W2, b2, W3, b3):
    full = lambda s: pl.BlockSpec(s, lambda i: (0, 0))
    return pl.pallas_call(
        _mlp_body,
        grid=(B // MLP_BLK,),
        in_specs=[
            pl.BlockSpec((MLP_BLK, L), lambda i: (i, 0)),
            full(W1.shape), full((1, L)),
            full(W2.shape), full((1, 2 * L)),
            full(W3.shape), full((1, H)),
        ],
        out_specs=pl.BlockSpec((MLP_BLK, H), lambda i: (i, 0)),
        out_shape=jax.ShapeDtypeStruct((B, H), jnp.float32),
    )(h, W1, b1.reshape(1, L), W2, b2.reshape(1, 2 * L), W3, b3.reshape(1, H))


def kernel(mof, emb, W1, b1, W2, b2, W3, b3):
    offs = (jnp.arange(F, dtype=jnp.int32) * V)[None, :]
    v = mof.astype(jnp.int32)
    flat_idx = ((v % VQ) * 8 + v // VQ + offs).reshape(-1, 128)
    emb_flat = _transpose_tables(emb.transpose(0, 2, 1))
    h = _gather_sum(emb_flat, flat_idx)
    return _mlp(h, W1, b1, W2, b2, W3, b3)
